# baseline (speedup 1.0000x reference)
import functools

import jax
import jax.numpy as jnp
from jax import lax
from jax.experimental import pallas as pl
from jax.experimental.pallas import tpu as pltpu
from jax.experimental.pallas import tpu_sc as plsc

BATCH = 4096
HIST_LEN = 50
HIST_PAD = 56
DIM = 64
DIM_PAD = 128

NUM_CORES = 2
NUM_SUBCORES = 16
NUM_WORKERS = NUM_CORES * NUM_SUBCORES
B_PER_WORKER = BATCH // NUM_WORKERS
CHUNK_B = 16
NUM_CHUNKS = B_PER_WORKER // CHUNK_B

_mesh = plsc.VectorSubcoreMesh(core_axis_name="c", subcore_axis_name="s")


@functools.partial(
    pl.kernel,
    mesh=_mesh,
    out_type=jax.ShapeDtypeStruct((BATCH, HIST_PAD, DIM_PAD), jnp.float32),
    scratch_types=[
        pltpu.VMEM((B_PER_WORKER, HIST_LEN), jnp.int32),
        pltpu.VMEM((CHUNK_B, HIST_LEN, DIM), jnp.float32),
        pltpu.VMEM((CHUNK_B, HIST_LEN, DIM), jnp.float32),
        pltpu.SemaphoreType.DMA,
        pltpu.SemaphoreType.DMA,
    ],
    compiler_params=pltpu.CompilerParams(use_tc_tiling_on_sc=False),
)
def _gather_kernel(idx_hbm, table_hbm, out_3d, idx_v, rows_a, rows_b, g_sem, s_sem):
    wid = lax.axis_index("s") * NUM_CORES + lax.axis_index("c")
    bbase = wid * B_PER_WORKER

    pltpu.sync_copy(idx_hbm.at[pl.ds(bbase, B_PER_WORKER)], idx_v)

    bufs = (rows_a, rows_b)
    gathers = [[None] * CHUNK_B, [None] * CHUNK_B]
    stores = [None, None]
    for i in range(NUM_CHUNKS + 1):
        b = i % 2
        if i < NUM_CHUNKS:
            if i >= 2:
                stores[b].wait()
            for j in range(CHUNK_B):
                gathers[b][j] = pltpu.async_copy(
                    table_hbm.at[idx_v.at[i * CHUNK_B + j]], bufs[b].at[j], g_sem)
        if i >= 1:
            pb = (i - 1) % 2
            for j in range(CHUNK_B):
                gathers[pb][j].wait()
            stores[pb] = pltpu.async_copy(
                bufs[pb],
                out_3d.at[pl.ds(bbase + (i - 1) * CHUNK_B, CHUNK_B),
                          pl.ds(0, HIST_LEN), pl.ds(0, DIM)], s_sem)
    stores[(NUM_CHUNKS - 2) % 2].wait()
    stores[(NUM_CHUNKS - 1) % 2].wait()


def kernel(input_ids, gpu_weight):
    out_pad = _gather_kernel(input_ids.astype(jnp.int32), gpu_weight)
    return out_pad[:, :HIST_LEN, :DIM]


# speedup vs baseline: 1.0055x; 1.0055x over previous
import functools

import jax
import jax.numpy as jnp
from jax import lax
from jax.experimental import pallas as pl
from jax.experimental.pallas import tpu as pltpu
from jax.experimental.pallas import tpu_sc as plsc

BATCH = 4096
HIST_LEN = 50
HIST_PAD = 56
DIM = 64
DIM_PAD = 128

NUM_CORES = 2
NUM_SUBCORES = 16
NUM_WORKERS = NUM_CORES * NUM_SUBCORES
B_PER_WORKER = BATCH // NUM_WORKERS
CHUNK_B = 8
NUM_BUFS = 3
NUM_CHUNKS = B_PER_WORKER // CHUNK_B

_mesh = plsc.VectorSubcoreMesh(core_axis_name="c", subcore_axis_name="s")


@functools.partial(
    pl.kernel,
    mesh=_mesh,
    out_type=jax.ShapeDtypeStruct((BATCH, HIST_PAD, DIM_PAD), jnp.float32),
    scratch_types=[
        pltpu.VMEM((B_PER_WORKER, HIST_LEN), jnp.int32),
        pltpu.VMEM((CHUNK_B, HIST_LEN, DIM), jnp.float32),
        pltpu.VMEM((CHUNK_B, HIST_LEN, DIM), jnp.float32),
        pltpu.VMEM((CHUNK_B, HIST_LEN, DIM), jnp.float32),
        pltpu.SemaphoreType.DMA,
        pltpu.SemaphoreType.DMA,
    ],
    compiler_params=pltpu.CompilerParams(use_tc_tiling_on_sc=False),
)
def _gather_kernel(idx_hbm, table_hbm, out_3d, idx_v, rows_a, rows_b, rows_c, g_sem, s_sem):
    wid = lax.axis_index("s") * NUM_CORES + lax.axis_index("c")
    bbase = wid * B_PER_WORKER

    pltpu.sync_copy(idx_hbm.at[pl.ds(bbase, B_PER_WORKER)], idx_v)

    bufs = (rows_a, rows_b, rows_c)
    gathers = [[None] * CHUNK_B for _ in range(NUM_BUFS)]
    stores = [[None] * CHUNK_B for _ in range(NUM_BUFS)]
    for i in range(NUM_CHUNKS + 1):
        b = i % NUM_BUFS
        if i < NUM_CHUNKS:
            for j in range(CHUNK_B):
                if i >= NUM_BUFS:
                    stores[b][j].wait()
                gathers[b][j] = pltpu.async_copy(
                    table_hbm.at[idx_v.at[i * CHUNK_B + j]], bufs[b].at[j], g_sem)
        if i >= 1:
            pb = (i - 1) % NUM_BUFS
            for j in range(CHUNK_B):
                gathers[pb][j].wait()
                stores[pb][j] = pltpu.async_copy(
                    bufs[pb].at[j],
                    out_3d.at[bbase + (i - 1) * CHUNK_B + j, pl.ds(0, HIST_LEN),
                              pl.ds(0, DIM)], s_sem)
    for j in range(CHUNK_B):
        for k in range(min(NUM_BUFS, NUM_CHUNKS)):
            stores[(NUM_CHUNKS - 1 - k) % NUM_BUFS][j].wait()


def kernel(input_ids, gpu_weight):
    out_pad = _gather_kernel(input_ids.astype(jnp.int32), gpu_weight)
    return out_pad[:, :HIST_LEN, :DIM]
